# baseline (speedup 1.0000x reference)
import functools

import jax
import jax.numpy as jnp
from jax import lax
from jax.experimental import pallas as pl
from jax.experimental.pallas import tpu as pltpu
from jax.experimental.pallas import tpu_sc as plsc

GAMMA_ = 0.99
LANES = 16
N_WORKERS = 16


def _splat(vec, i):
    ci = jnp.full((LANES,), i, jnp.int32)
    return vec.at[ci].get(mode="promise_in_bounds")


def _lane_total(vec):
    return _splat(plsc.cumsum(vec), LANES - 1)


def _sc_body(consts, psi_hbm, tgt_hbm, feat_hbm, a0_hbm, a1_hbm, slb_hbm,
             sl_hbm, out_hbm, va0, va1, vslb, vsl, vpi, vti, vfi, prow,
             trow, frow, part_ref, shared, gath, outv, sem):
    L, V, n_rows, ipw = consts
    c = lax.axis_index("c")
    s = lax.axis_index("s")

    @pl.when(c == 0)
    def _work():
        base = s * ipw
        b = lax.div(base, L)
        t0 = lax.rem(base, L)
        pltpu.sync_copy(a0_hbm.at[pl.ds(base, LANES)], va0)
        pltpu.sync_copy(a1_hbm.at[pl.ds(base, LANES)], va1)
        pltpu.sync_copy(slb_hbm.at[pl.ds(base, LANES)], vslb)
        t = lax.iota(jnp.int32, LANES) + t0
        av0 = va0[...]
        av1 = va1[...]
        slb = vslb[...]
        row_base = b * (L * V)
        vpi[...] = row_base + t * V + av0
        vti[...] = jnp.minimum(row_base + (t + 1) * V + av1, n_rows - 1)
        vfi[...] = b * (L + 1) + (t + 1)
        is_term = t == slb - 1
        cf_vec = jnp.where(is_term, 1.0, 0.0).astype(jnp.float32)
        cg_vec = jnp.where((t < L - 2) & jnp.logical_not(is_term),
                           GAMMA_, 0.0).astype(jnp.float32)
        vm_vec = jnp.where(t <= L - 2, 1.0, 0.0).astype(jnp.float32)
        cp1 = pltpu.async_copy(psi_hbm.at[vpi], prow, sem)
        cp2 = pltpu.async_copy(tgt_hbm.at[vti], trow, sem)
        cp3 = pltpu.async_copy(feat_hbm.at[vfi], frow, sem)
        cp1.wait()
        cp2.wait()
        cp3.wait()
        acc = jnp.zeros((LANES,), jnp.float32)
        for i in range(ipw):
            cfs = _splat(cf_vec, i)
            cgs = _splat(cg_vec, i)
            vms = _splat(vm_vec, i)
            for k in range(prow.shape[1] // LANES):
                sl_ = pl.ds(k * LANES, LANES)
                d = (vms * prow[i, sl_] - cgs * trow[i, sl_]
                     - cfs * frow[i, sl_])
                acc = acc + d * d
        part_ref[...] = acc
        pltpu.sync_copy(part_ref, shared.at[s])
        plsc.subcore_barrier()

        @pl.when(s == 0)
        def _reduce():
            pltpu.sync_copy(sl_hbm, vsl)
            pltpu.sync_copy(shared, gath)
            tot = gath[0, :]
            for i in range(1, N_WORKERS):
                tot = tot + gath[i, :]
            total = _lane_total(tot)
            denom = _lane_total(vsl[...].astype(jnp.float32))
            outv[...] = total / denom
            pltpu.sync_copy(outv, out_hbm)


def kernel(psi, target_psi, actions, features, seq_lens):
    B, L, V, F = psi.shape
    n_rows = B * L * V
    ipw = (B * L) // N_WORKERS
    assert (B * L) % N_WORKERS == 0 and L % ipw == 0 and ipw == LANES
    assert F % LANES == 0

    psi2 = psi.reshape(n_rows, F)
    tgt2 = target_psi.reshape(n_rows, F)
    feat2 = features.reshape(B * (L + 1), F)
    a = actions.astype(jnp.int32)
    a0 = jnp.pad(a, ((0, 0), (0, 1))).reshape(-1)
    a1 = jnp.pad(a[:, 1:], ((0, 0), (0, 2))).reshape(-1)
    sl = seq_lens.astype(jnp.int32)
    slb = jnp.repeat(sl, L)
    sl16 = jnp.zeros((LANES,), jnp.int32).at[:B].set(sl)

    mesh = plsc.VectorSubcoreMesh(core_axis_name="c", subcore_axis_name="s")
    run = pl.kernel(
        functools.partial(_sc_body, (L, V, n_rows, ipw)),
        out_type=jax.ShapeDtypeStruct((LANES,), jnp.float32),
        mesh=mesh,
        compiler_params=pltpu.CompilerParams(
            use_tc_tiling_on_sc=False, needs_layout_passes=False),
        scratch_types=[
            pltpu.VMEM((LANES,), jnp.int32),
            pltpu.VMEM((LANES,), jnp.int32),
            pltpu.VMEM((LANES,), jnp.int32),
            pltpu.VMEM((LANES,), jnp.int32),
            pltpu.VMEM((LANES,), jnp.int32),
            pltpu.VMEM((LANES,), jnp.int32),
            pltpu.VMEM((LANES,), jnp.int32),
            pltpu.VMEM((LANES, F), jnp.float32),
            pltpu.VMEM((LANES, F), jnp.float32),
            pltpu.VMEM((LANES, F), jnp.float32),
            pltpu.VMEM((LANES,), jnp.float32),
            pltpu.VMEM_SHARED((N_WORKERS, LANES), jnp.float32),
            pltpu.VMEM((N_WORKERS, LANES), jnp.float32),
            pltpu.VMEM((LANES,), jnp.float32),
            pltpu.SemaphoreType.DMA,
        ],
    )
    out = run(psi2, tgt2, feat2, a0, a1, slb, sl16)
    return out[0]


# speedup vs baseline: 7.9182x; 7.9182x over previous
import functools

import jax
import jax.numpy as jnp
from jax import lax
from jax.experimental import pallas as pl
from jax.experimental.pallas import tpu as pltpu
from jax.experimental.pallas import tpu_sc as plsc

GAMMA_ = 0.99
LANES = 16
N_WORKERS = 16
SUB = 8
LN = 128


def _splat(vec, i):
    ci = jnp.full((LANES,), i, jnp.int32)
    return vec.at[ci].get(mode="promise_in_bounds")


def _lane_total(vec):
    return _splat(plsc.cumsum(vec), LANES - 1)


def _sc_body(consts, zpsi_hbm, ztgt_hbm, feat_hbm, a0_hbm, a1_hbm, slb_hbm,
             sl_hbm, out_hbm, va0, va1, vslb, vsl, vfi, vkp0, vkp1, vkt0,
             vkt1, pt0, pt1, tt0, tt1, frow, part_ref, shared, gath, outv,
             semf, semp0, semp1, semt0, semt1):
    L, n_tiles, ipw = consts
    c = lax.axis_index("c")
    s = lax.axis_index("s")

    @pl.when(c == 0)
    def _work():
        base = s * ipw
        b = lax.div(base, L)
        pltpu.sync_copy(a0_hbm.at[pl.ds(base, LANES)], va0)
        pltpu.sync_copy(a1_hbm.at[pl.ds(base, LANES)], va1)
        pltpu.sync_copy(slb_hbm.at[pl.ds(base, LANES)], vslb)
        lane = lax.iota(jnp.int32, LANES)
        t = lane + lax.rem(base, L)
        av0 = va0[...]
        av1 = va1[...]
        slb = vslb[...]
        vfi[...] = b * (L + 1) + (t + 1)
        cpf = pltpu.async_copy(feat_hbm.at[vfi], frow, semf)
        vt0 = lax.div(av0, LN)
        vl0 = lax.rem(av0, LN)
        vt1 = lax.div(av1, LN)
        vl1 = lax.rem(av1, LN)
        bl = base + lane
        blt = bl + 1
        is_term = t == slb - 1
        cf_vec = jnp.where(is_term, 1.0, 0.0).astype(jnp.float32)
        cg_vec = jnp.where((t < L - 2) & jnp.logical_not(is_term),
                           GAMMA_, 0.0).astype(jnp.float32)
        vm_vec = jnp.where(t <= L - 2, 1.0, 0.0).astype(jnp.float32)
        ftv = lax.rem(lane, SUB)

        def fire(p, vkp, vkt, ptiles, ttiles, semp, semt):
            i0, i1 = 2 * p, 2 * p + 1
            blA, blB = _splat(bl, i0), _splat(bl, i1)
            vtA, vtB = _splat(vt0, i0), _splat(vt0, i1)
            lo = lane < SUB
            vkp[...] = (jnp.where(lo, blA, blB) * 64 + ftv * SUB
                        + jnp.where(lo, vtA, vtB))
            btA, btB = _splat(blt, i0), _splat(blt, i1)
            wtA, wtB = _splat(vt1, i0), _splat(vt1, i1)
            vkt[...] = jnp.minimum(
                jnp.where(lo, btA, btB) * 64 + ftv * SUB
                + jnp.where(lo, wtA, wtB), n_tiles - 1)
            cp = pltpu.async_copy(zpsi_hbm.at[vkp], ptiles, semp)
            ct = pltpu.async_copy(ztgt_hbm.at[vkt], ttiles, semt)
            return cp, ct

        slots = ((vkp0, vkt0, pt0, tt0, semp0, semt0),
                 (vkp1, vkt1, pt1, tt1, semp1, semt1))
        n_pairs = ipw // 2
        pend = {0: fire(0, *slots[0])}
        cpf.wait()
        acc = jnp.zeros((LANES,), jnp.float32)
        jft = lax.div(lane, SUB)
        fsc = lax.rem(lane, SUB)
        for p in range(n_pairs):
            if p + 1 < n_pairs:
                pend[p + 1] = fire(p + 1, *slots[(p + 1) % 2])
            cp, ct = pend.pop(p)
            cp.wait()
            ct.wait()
            _, _, ptiles, ttiles, _, _ = slots[p % 2]
            for m in range(2):
                i = 2 * p + m
                vls = _splat(vl0, i)
                vlt = _splat(vl1, i)
                cfs = _splat(cf_vec, i)
                cgs = _splat(cg_vec, i)
                vms = _splat(vm_vec, i)
                for k in range(4):
                    j0 = jft + (m * SUB + 2 * k)
                    pg = plsc.load_gather(ptiles, [j0, fsc, vls])
                    tg = plsc.load_gather(ttiles, [j0, fsc, vlt])
                    fr = frow[i, pl.ds(k * LANES, LANES)]
                    d = vms * pg - cgs * tg - cfs * fr
                    acc = acc + d * d
        part_ref[...] = acc
        pltpu.sync_copy(part_ref, shared.at[s])
        plsc.subcore_barrier()

        @pl.when(s == 0)
        def _reduce():
            pltpu.sync_copy(sl_hbm, vsl)
            pltpu.sync_copy(shared, gath)
            tot = gath[0, :]
            for i in range(1, N_WORKERS):
                tot = tot + gath[i, :]
            total = _lane_total(tot)
            denom = _lane_total(vsl[...].astype(jnp.float32))
            outv[...] = total / denom
            pltpu.sync_copy(outv, out_hbm)


def kernel(psi, target_psi, actions, features, seq_lens):
    B, L, V, F = psi.shape
    n_tiles = B * L * (F // SUB) * (V // LN)
    ipw = (B * L) // N_WORKERS
    assert (B * L) % N_WORKERS == 0 and L % ipw == 0 and ipw == LANES
    assert F == 64 and V // LN == SUB

    def tile_view(x):
        return (x.reshape(B, L, V // LN, LN, F // SUB, SUB)
                 .transpose(0, 1, 4, 2, 5, 3)
                 .reshape(n_tiles, SUB, LN))

    zpsi = tile_view(psi)
    ztgt = tile_view(target_psi)
    feat2 = features.reshape(B * (L + 1), F)
    a = actions.astype(jnp.int32)
    a0 = jnp.pad(a, ((0, 0), (0, 1))).reshape(-1)
    a1 = jnp.pad(a[:, 1:], ((0, 0), (0, 2))).reshape(-1)
    sl = seq_lens.astype(jnp.int32)
    slb = jnp.repeat(sl, L)
    sl16 = jnp.zeros((LANES,), jnp.int32).at[:B].set(sl)

    mesh = plsc.VectorSubcoreMesh(core_axis_name="c", subcore_axis_name="s")
    run = pl.kernel(
        functools.partial(_sc_body, (L, n_tiles, ipw)),
        out_type=jax.ShapeDtypeStruct((LANES,), jnp.float32),
        mesh=mesh,
        compiler_params=pltpu.CompilerParams(
            use_tc_tiling_on_sc=False, needs_layout_passes=False),
        scratch_types=[
            pltpu.VMEM((LANES,), jnp.int32),
            pltpu.VMEM((LANES,), jnp.int32),
            pltpu.VMEM((LANES,), jnp.int32),
            pltpu.VMEM((LANES,), jnp.int32),
            pltpu.VMEM((LANES,), jnp.int32),
            pltpu.VMEM((LANES,), jnp.int32),
            pltpu.VMEM((LANES,), jnp.int32),
            pltpu.VMEM((LANES,), jnp.int32),
            pltpu.VMEM((LANES,), jnp.int32),
            pltpu.VMEM((LANES, SUB, LN), jnp.float32),
            pltpu.VMEM((LANES, SUB, LN), jnp.float32),
            pltpu.VMEM((LANES, SUB, LN), jnp.float32),
            pltpu.VMEM((LANES, SUB, LN), jnp.float32),
            pltpu.VMEM((LANES, F), jnp.float32),
            pltpu.VMEM((LANES,), jnp.float32),
            pltpu.VMEM_SHARED((N_WORKERS, LANES), jnp.float32),
            pltpu.VMEM((N_WORKERS, LANES), jnp.float32),
            pltpu.VMEM((LANES,), jnp.float32),
            pltpu.SemaphoreType.DMA,
            pltpu.SemaphoreType.DMA,
            pltpu.SemaphoreType.DMA,
            pltpu.SemaphoreType.DMA,
            pltpu.SemaphoreType.DMA,
        ],
    )
    out = run(zpsi, ztgt, feat2, a0, a1, slb, sl16)
    return out[0]


# speedup vs baseline: 10.2870x; 1.2992x over previous
import functools

import jax
import jax.numpy as jnp
from jax import lax
from jax.experimental import pallas as pl
from jax.experimental.pallas import tpu as pltpu
from jax.experimental.pallas import tpu_sc as plsc

GAMMA_ = 0.99
LANES = 16
N_WORKERS = 16
SUB = 8
LN = 128
GRAN = 16


def _splat(vec, i):
    ci = jnp.full((LANES,), i, jnp.int32)
    return vec.at[ci].get(mode="promise_in_bounds")


def _lane_total(vec):
    return _splat(plsc.cumsum(vec), LANES - 1)


def _sc_body(consts, zpsi_hbm, ztgt_hbm, feat_hbm, a0_hbm, a1_hbm, slb_hbm,
             sl_hbm, out_hbm, va0, va1, vslb, vsl, vfi, qp, qt, pdst, tdst,
             frow, part_ref, shared, gath, outv, semf, semg):
    L, n_gran, ipw = consts
    c = lax.axis_index("c")
    s = lax.axis_index("s")

    @pl.when(c == 0)
    def _work():
        base = s * ipw
        b = lax.div(base, L)
        pltpu.sync_copy(a0_hbm.at[pl.ds(base, LANES)], va0)
        pltpu.sync_copy(a1_hbm.at[pl.ds(base, LANES)], va1)
        pltpu.sync_copy(slb_hbm.at[pl.ds(base, LANES)], vslb)
        lane = lax.iota(jnp.int32, LANES)
        t = lane + lax.rem(base, L)
        av0 = va0[...]
        av1 = va1[...]
        slb = vslb[...]
        vfi[...] = b * (L + 1) + (t + 1)
        cpf = pltpu.async_copy(feat_hbm.at[vfi], frow, semf)
        sq0 = (base + lane) * (64 * 64) + lax.div(av0, LN) * 64 \
            + lax.div(lax.rem(av0, LN), GRAN)
        sq1 = (base + lane + 1) * (64 * 64) + lax.div(av1, LN) * 64 \
            + lax.div(lax.rem(av1, LN), GRAN)
        vlm0 = lax.rem(av0, GRAN)
        vlm1 = lax.rem(av1, GRAN)
        is_term = t == slb - 1
        cf_vec = jnp.where(is_term, 1.0, 0.0).astype(jnp.float32)
        cg_vec = jnp.where((t < L - 2) & jnp.logical_not(is_term),
                           GAMMA_, 0.0).astype(jnp.float32)
        vm_vec = jnp.where(t <= L - 2, 1.0, 0.0).astype(jnp.float32)
        offs = [lax.div(jnp.int32(16 * cc) + lane, SUB) * 512
                + lax.rem(jnp.int32(16 * cc) + lane, SUB) * 8
                for cc in range(4)]
        for i in range(ipw):
            sp0 = _splat(sq0, i)
            sp1 = _splat(sq1, i)
            for cc in range(4):
                qp[i, pl.ds(cc * LANES, LANES)] = sp0 + offs[cc]
                qt[i, pl.ds(cc * LANES, LANES)] = jnp.minimum(
                    sp1 + offs[cc], n_gran - 1)
        cps = []
        for i in range(ipw):
            cps.append(pltpu.async_copy(zpsi_hbm.at[qp.at[i]],
                                        pdst.at[i], semg))
            cps.append(pltpu.async_copy(ztgt_hbm.at[qt.at[i]],
                                        tdst.at[i], semg))
        cpf.wait()
        for cp in cps:
            cp.wait()
        acc = jnp.zeros((LANES,), jnp.float32)
        for i in range(ipw):
            vls = _splat(vlm0, i)
            vlt = _splat(vlm1, i)
            cfs = _splat(cf_vec, i)
            cgs = _splat(cg_vec, i)
            vms = _splat(vm_vec, i)
            ii = jnp.full((LANES,), i, jnp.int32)
            for k in range(4):
                rows = lane + (k * LANES)
                pg = plsc.load_gather(pdst, [ii, rows, vls])
                tg = plsc.load_gather(tdst, [ii, rows, vlt])
                fr = frow[i, pl.ds(k * LANES, LANES)]
                d = vms * pg - cgs * tg - cfs * fr
                acc = acc + d * d
        part_ref[...] = acc
        pltpu.sync_copy(part_ref, shared.at[s])
        plsc.subcore_barrier()

        @pl.when(s == 0)
        def _reduce():
            pltpu.sync_copy(sl_hbm, vsl)
            pltpu.sync_copy(shared, gath)
            tot = gath[0, :]
            for i in range(1, N_WORKERS):
                tot = tot + gath[i, :]
            total = _lane_total(tot)
            denom = _lane_total(vsl[...].astype(jnp.float32))
            outv[...] = total / denom
            pltpu.sync_copy(outv, out_hbm)


def kernel(psi, target_psi, actions, features, seq_lens):
    B, L, V, F = psi.shape
    n_gran = B * L * (F // SUB) * (V // LN) * SUB * (LN // GRAN)
    ipw = (B * L) // N_WORKERS
    assert (B * L) % N_WORKERS == 0 and L % ipw == 0 and ipw == LANES
    assert F == 64 and V // LN == SUB

    def gran_view(x):
        return (x.reshape(B, L, V // LN, LN, F // SUB, SUB)
                 .transpose(0, 1, 4, 2, 5, 3)
                 .reshape(n_gran, GRAN))

    zpsi = gran_view(psi)
    ztgt = gran_view(target_psi)
    feat2 = features.reshape(B * (L + 1), F)
    a = actions.astype(jnp.int32)
    a0 = jnp.pad(a, ((0, 0), (0, 1))).reshape(-1)
    a1 = jnp.pad(a[:, 1:], ((0, 0), (0, 2))).reshape(-1)
    sl = seq_lens.astype(jnp.int32)
    slb = jnp.repeat(sl, L)
    sl16 = jnp.zeros((LANES,), jnp.int32).at[:B].set(sl)

    mesh = plsc.VectorSubcoreMesh(core_axis_name="c", subcore_axis_name="s")
    run = pl.kernel(
        functools.partial(_sc_body, (L, n_gran, ipw)),
        out_type=jax.ShapeDtypeStruct((LANES,), jnp.float32),
        mesh=mesh,
        compiler_params=pltpu.CompilerParams(
            use_tc_tiling_on_sc=False, needs_layout_passes=False),
        scratch_types=[
            pltpu.VMEM((LANES,), jnp.int32),
            pltpu.VMEM((LANES,), jnp.int32),
            pltpu.VMEM((LANES,), jnp.int32),
            pltpu.VMEM((LANES,), jnp.int32),
            pltpu.VMEM((LANES,), jnp.int32),
            pltpu.VMEM((LANES, 64), jnp.int32),
            pltpu.VMEM((LANES, 64), jnp.int32),
            pltpu.VMEM((LANES, 64, GRAN), jnp.float32),
            pltpu.VMEM((LANES, 64, GRAN), jnp.float32),
            pltpu.VMEM((LANES, F), jnp.float32),
            pltpu.VMEM((LANES,), jnp.float32),
            pltpu.VMEM_SHARED((N_WORKERS, LANES), jnp.float32),
            pltpu.VMEM((N_WORKERS, LANES), jnp.float32),
            pltpu.VMEM((LANES,), jnp.float32),
            pltpu.SemaphoreType.DMA,
            pltpu.SemaphoreType.DMA,
        ],
    )
    out = run(zpsi, ztgt, feat2, a0, a1, slb, sl16)
    return out[0]


# speedup vs baseline: 10.8082x; 1.0507x over previous
import functools

import jax
import jax.numpy as jnp
from jax import lax
from jax.experimental import pallas as pl
from jax.experimental.pallas import tpu as pltpu
from jax.experimental.pallas import tpu_sc as plsc

GAMMA_ = 0.99
LANES = 16
N_WORKERS = 16
SUB = 8
LN = 128
GRAN = 16


def _splat(vec, i):
    ci = jnp.full((LANES,), i, jnp.int32)
    return vec.at[ci].get(mode="promise_in_bounds")


def _lane_total(vec):
    return _splat(plsc.cumsum(vec), LANES - 1)


def _sc_body(consts, zpsi_hbm, ztgt_hbm, feat_hbm, a0_hbm, a1_hbm, slb_hbm,
             sl_hbm, out_hbm, va0, va1, vslb, vsl, vfi, qp, qt, pdst, tdst,
             frow, part_ref, shared, gath, outv, semf, semg):
    L, n_gran, ipw = consts
    c = lax.axis_index("c")
    s = lax.axis_index("s")

    @pl.when(c == 0)
    def _work():
        base = s * ipw
        b = lax.div(base, L)
        pltpu.sync_copy(a0_hbm.at[pl.ds(base, LANES)], va0)
        pltpu.sync_copy(a1_hbm.at[pl.ds(base, LANES)], va1)
        pltpu.sync_copy(slb_hbm.at[pl.ds(base, LANES)], vslb)
        lane = lax.iota(jnp.int32, LANES)
        t = lane + lax.rem(base, L)
        av0 = va0[...]
        av1 = va1[...]
        slb = vslb[...]
        vfi[...] = b * (L + 1) + (t + 1)
        cpf = pltpu.async_copy(feat_hbm.at[vfi], frow, semf)
        sq0 = (base + lane) * (64 * 64) + lax.div(av0, LN) * 64 \
            + lax.div(lax.rem(av0, LN), GRAN)
        sq1 = (base + lane + 1) * (64 * 64) + lax.div(av1, LN) * 64 \
            + lax.div(lax.rem(av1, LN), GRAN)
        vlm0 = lax.rem(av0, GRAN)
        vlm1 = lax.rem(av1, GRAN)
        is_term = t == slb - 1
        cf_vec = jnp.where(is_term, 1.0, 0.0).astype(jnp.float32)
        cg_vec = jnp.where((t < L - 2) & jnp.logical_not(is_term),
                           GAMMA_, 0.0).astype(jnp.float32)
        vm_vec = jnp.where(t <= L - 2, 1.0, 0.0).astype(jnp.float32)
        offs = [lax.div(jnp.int32(16 * cc) + lane, SUB) * 512
                + lax.rem(jnp.int32(16 * cc) + lane, SUB) * 8
                for cc in range(4)]
        for i in range(ipw):
            sp0 = _splat(sq0, i)
            sp1 = _splat(sq1, i)
            for cc in range(4):
                qp[i, pl.ds(cc * LANES, LANES)] = sp0 + offs[cc]
                qt[i, pl.ds(cc * LANES, LANES)] = jnp.minimum(
                    sp1 + offs[cc], n_gran - 1)
        cps = []
        for i in range(ipw):
            cps.append(pltpu.async_copy(zpsi_hbm.at[qp.at[i]],
                                        pdst.at[i], semg))
            cps.append(pltpu.async_copy(ztgt_hbm.at[qt.at[i]],
                                        tdst.at[i], semg))
        cpf.wait()
        for cp in cps:
            cp.wait()
        acc = jnp.zeros((LANES,), jnp.float32)
        for i in range(ipw):
            vls = _splat(vlm0, i)
            vlt = _splat(vlm1, i)
            cfs = _splat(cf_vec, i)
            cgs = _splat(cg_vec, i)
            vms = _splat(vm_vec, i)
            ii = jnp.full((LANES,), i, jnp.int32)
            for k in range(4):
                rows = lane + (k * LANES)
                pg = plsc.load_gather(pdst, [ii, rows, vls])
                tg = plsc.load_gather(tdst, [ii, rows, vlt])
                fr = frow[i, pl.ds(k * LANES, LANES)]
                d = vms * pg - cgs * tg - cfs * fr
                acc = acc + d * d
        part_ref[...] = acc
        pltpu.sync_copy(part_ref, shared.at[s])
        plsc.subcore_barrier()

        @pl.when(s == 0)
        def _reduce():
            pltpu.sync_copy(sl_hbm, vsl)
            pltpu.sync_copy(shared, gath)
            tot = gath[0, :]
            for i in range(1, N_WORKERS):
                tot = tot + gath[i, :]
            total = _lane_total(tot)
            denom = _lane_total(vsl[...].astype(jnp.float32))
            outv[...] = total / denom
            pltpu.sync_copy(outv, out_hbm)


def kernel(psi, target_psi, actions, features, seq_lens):
    B, L, V, F = psi.shape
    n_gran = B * L * (F // SUB) * (V // LN) * SUB * (LN // GRAN)
    ipw = (B * L) // N_WORKERS
    assert (B * L) % N_WORKERS == 0 and L % ipw == 0 and ipw == LANES
    assert F == 64 and V // LN == SUB

    def gran_view(x):
        return (x.reshape(B, L, V // LN, LN, F // SUB, SUB)
                 .transpose(0, 1, 4, 2, 5, 3)
                 .reshape(n_gran, GRAN))

    zpsi = gran_view(psi)
    ztgt = gran_view(target_psi)
    feat2 = features.reshape(B * (L + 1), F)
    a = actions.astype(jnp.int32)
    a0 = jnp.pad(a, ((0, 0), (0, 1))).reshape(-1)
    a1 = jnp.pad(a[:, 1:], ((0, 0), (0, 2))).reshape(-1)
    sl = seq_lens.astype(jnp.int32)
    slb = jnp.repeat(sl, L)
    sl16 = jnp.zeros((LANES,), jnp.int32).at[:B].set(sl)

    mesh = plsc.VectorSubcoreMesh(
        core_axis_name="c", subcore_axis_name="s", num_cores=1)
    run = pl.kernel(
        functools.partial(_sc_body, (L, n_gran, ipw)),
        out_type=jax.ShapeDtypeStruct((LANES,), jnp.float32),
        mesh=mesh,
        compiler_params=pltpu.CompilerParams(
            use_tc_tiling_on_sc=False, needs_layout_passes=False),
        scratch_types=[
            pltpu.VMEM((LANES,), jnp.int32),
            pltpu.VMEM((LANES,), jnp.int32),
            pltpu.VMEM((LANES,), jnp.int32),
            pltpu.VMEM((LANES,), jnp.int32),
            pltpu.VMEM((LANES,), jnp.int32),
            pltpu.VMEM((LANES, 64), jnp.int32),
            pltpu.VMEM((LANES, 64), jnp.int32),
            pltpu.VMEM((LANES, 64, GRAN), jnp.float32),
            pltpu.VMEM((LANES, 64, GRAN), jnp.float32),
            pltpu.VMEM((LANES, F), jnp.float32),
            pltpu.VMEM((LANES,), jnp.float32),
            pltpu.VMEM_SHARED((N_WORKERS, LANES), jnp.float32),
            pltpu.VMEM((N_WORKERS, LANES), jnp.float32),
            pltpu.VMEM((LANES,), jnp.float32),
            pltpu.SemaphoreType.DMA,
            pltpu.SemaphoreType.DMA,
        ],
    )
    out = run(zpsi, ztgt, feat2, a0, a1, slb, sl16)
    return out[0]


# speedup vs baseline: 12.6295x; 1.1685x over previous
import functools

import jax
import jax.numpy as jnp
from jax import lax
from jax.experimental import pallas as pl
from jax.experimental.pallas import tpu as pltpu
from jax.experimental.pallas import tpu_sc as plsc

GAMMA_ = 0.99
LANES = 16
N_WORKERS = 16
SUB = 8
LN = 128
GRAN = 16


def _splat(vec, i):
    ci = jnp.full((LANES,), i, jnp.int32)
    return vec.at[ci].get(mode="promise_in_bounds")


def _lane_total(vec):
    return _splat(plsc.cumsum(vec), LANES - 1)


def _sc_body(consts, zpsi_hbm, ztgt_hbm, feat_hbm, act_hbm, sl_hbm,
             out_hbm, vact, vsl4, vfi, qp, qt, pdst, tdst, frow, part_ref,
             shared, gath, outv, semf, semg):
    B, L, n_gran, ipw = consts
    s = lax.axis_index("s")
    base = s * ipw
    b = lax.div(base, L)
    pltpu.sync_copy(act_hbm, vact)
    pltpu.sync_copy(sl_hbm, vsl4)
    lane = lax.iota(jnp.int32, LANES)
    t = lane + lax.rem(base, L)
    bsp = lane * 0 + b
    av0 = plsc.load_gather(vact, [bsp, jnp.minimum(t, L - 2)])
    av1 = plsc.load_gather(vact, [bsp, jnp.minimum(t + 1, L - 2)])
    slb = plsc.load_gather(vsl4, [bsp])
    vfi[...] = b * (L + 1) + (t + 1)
    cpf = pltpu.async_copy(feat_hbm.at[vfi], frow, semf)
    sq0 = (base + lane) * (64 * 64) + lax.div(av0, LN) * 64 \
        + lax.div(lax.rem(av0, LN), GRAN)
    sq1 = (base + lane + 1) * (64 * 64) + lax.div(av1, LN) * 64 \
        + lax.div(lax.rem(av1, LN), GRAN)
    vlm0 = lax.rem(av0, GRAN)
    vlm1 = lax.rem(av1, GRAN)
    is_term = t == slb - 1
    cf_vec = jnp.where(is_term, 1.0, 0.0).astype(jnp.float32)
    cg_vec = jnp.where((t < L - 2) & jnp.logical_not(is_term),
                       GAMMA_, 0.0).astype(jnp.float32)
    vm_vec = jnp.where(t <= L - 2, 1.0, 0.0).astype(jnp.float32)
    offs = [lax.div(jnp.int32(16 * cc) + lane, SUB) * 512
            + lax.rem(jnp.int32(16 * cc) + lane, SUB) * 8
            for cc in range(4)]
    for i in range(ipw):
        sp0 = _splat(sq0, i)
        sp1 = _splat(sq1, i)
        for cc in range(4):
            qp[i, pl.ds(cc * LANES, LANES)] = sp0 + offs[cc]
            qt[i, pl.ds(cc * LANES, LANES)] = jnp.minimum(
                sp1 + offs[cc], n_gran - 1)
    cps = []
    for i in range(ipw):
        cps.append(pltpu.async_copy(zpsi_hbm.at[qp.at[i]],
                                    pdst.at[i], semg))
        cps.append(pltpu.async_copy(ztgt_hbm.at[qt.at[i]],
                                    tdst.at[i], semg))
    cpf.wait()
    for cp in cps:
        cp.wait()
    acc = jnp.zeros((LANES,), jnp.float32)
    for i in range(ipw):
        vls = _splat(vlm0, i)
        vlt = _splat(vlm1, i)
        cfs = _splat(cf_vec, i)
        cgs = _splat(cg_vec, i)
        vms = _splat(vm_vec, i)
        ii = jnp.full((LANES,), i, jnp.int32)
        for k in range(4):
            rows = lane + (k * LANES)
            pg = plsc.load_gather(pdst, [ii, rows, vls])
            tg = plsc.load_gather(tdst, [ii, rows, vlt])
            fr = frow[i, pl.ds(k * LANES, LANES)]
            d = vms * pg - cgs * tg - cfs * fr
            acc = acc + d * d
    part_ref[...] = acc
    pltpu.sync_copy(part_ref, shared.at[s])
    plsc.subcore_barrier()

    @pl.when(s == 0)
    def _reduce():
        pltpu.sync_copy(shared, gath)
        tot = gath[0, :]
        for i in range(1, N_WORKERS):
            tot = tot + gath[i, :]
        total = _lane_total(tot)
        slv = plsc.load_gather(vsl4, [jnp.minimum(lane, B - 1)])
        slm = jnp.where(lane < B, slv, 0).astype(jnp.float32)
        denom = _lane_total(slm)
        outv[...] = total / denom
        pltpu.sync_copy(outv, out_hbm)


def kernel(psi, target_psi, actions, features, seq_lens):
    B, L, V, F = psi.shape
    n_gran = B * L * (F // SUB) * (V // LN) * SUB * (LN // GRAN)
    ipw = (B * L) // N_WORKERS
    assert (B * L) % N_WORKERS == 0 and L % ipw == 0 and ipw == LANES
    assert F == 64 and V // LN == SUB

    def gran_view(x):
        return (x.reshape(B, L, V // LN, LN, F // SUB, SUB)
                 .transpose(0, 1, 4, 2, 5, 3)
                 .reshape(n_gran, GRAN))

    zpsi = gran_view(psi)
    ztgt = gran_view(target_psi)
    feat2 = features.reshape(B * (L + 1), F)
    act = actions.astype(jnp.int32)
    sl4 = seq_lens.astype(jnp.int32)

    mesh = plsc.VectorSubcoreMesh(
        core_axis_name="c", subcore_axis_name="s", num_cores=1)
    run = pl.kernel(
        functools.partial(_sc_body, (B, L, n_gran, ipw)),
        out_type=jax.ShapeDtypeStruct((LANES,), jnp.float32),
        mesh=mesh,
        compiler_params=pltpu.CompilerParams(
            use_tc_tiling_on_sc=False, needs_layout_passes=False),
        scratch_types=[
            pltpu.VMEM((B, L - 1), jnp.int32),
            pltpu.VMEM((B,), jnp.int32),
            pltpu.VMEM((LANES,), jnp.int32),
            pltpu.VMEM((LANES, 64), jnp.int32),
            pltpu.VMEM((LANES, 64), jnp.int32),
            pltpu.VMEM((LANES, 64, GRAN), jnp.float32),
            pltpu.VMEM((LANES, 64, GRAN), jnp.float32),
            pltpu.VMEM((LANES, F), jnp.float32),
            pltpu.VMEM((LANES,), jnp.float32),
            pltpu.VMEM_SHARED((N_WORKERS, LANES), jnp.float32),
            pltpu.VMEM((N_WORKERS, LANES), jnp.float32),
            pltpu.VMEM((LANES,), jnp.float32),
            pltpu.SemaphoreType.DMA,
            pltpu.SemaphoreType.DMA,
        ],
    )
    out = run(zpsi, ztgt, feat2, act, sl4)
    return out[0]
